# baseline (speedup 1.0000x reference)
import functools

import jax
import jax.numpy as jnp
from jax import lax
from jax.experimental import pallas as pl
from jax.experimental.pallas import tpu as pltpu
from jax.experimental.pallas import tpu_sc as plsc

N = 10000
IN = 128
EMB = 16
H1 = 8
SLOPE = 0.2

NC = 2
NS = 16
NW = NC * NS
K = 64
RPT = 313
NP = NW * RPT

_mesh = plsc.VectorSubcoreMesh(
    core_axis_name="c", subcore_axis_name="s", num_cores=NC, num_subcores=NS)


def _leaky(v):
    return jnp.where(v > 0, v, v * SLOPE)


_GDN = lax.GatherDimensionNumbers(
    offset_dims=(), collapsed_slice_dims=(0,), start_index_map=(0,))


def _permute(vec, idx16):
    return lax.gather(vec, idx16[:, None], _GDN, (1,),
                      mode=lax.GatherScatterMode.PROMISE_IN_BOUNDS)


def _tc1_body(x_ref, w1_ref, as_ref, ad_ref, h1_ref, als_ref, ald_ref):
    h = jnp.dot(x_ref[...], w1_ref[...], preferred_element_type=jnp.float32)
    h1_ref[...] = h
    als_ref[...] = jnp.dot(h, as_ref[...], preferred_element_type=jnp.float32)
    ald_ref[...] = jnp.dot(h, ad_ref[...], preferred_element_type=jnp.float32)


def _tc2_body(accm_ref, h1_ref, als_ref, ald_ref, r8_ref, r_ref, b1_ref,
              w2_ref, as2_ref, ad2_ref, h2_ref, als2_ref, ald2_ref):
    t = accm_ref[0] + accm_ref[1]
    m = t[:, :IN]
    d8 = t[:, IN:]
    a_self = jnp.exp(_leaky(als_ref[...] + ald_ref[...]))
    den = (jnp.dot(d8, r8_ref[...], preferred_element_type=jnp.float32)
           + jnp.dot(a_self, r_ref[...], preferred_element_type=jnp.float32))
    a_bc = jnp.dot(a_self, r_ref[...], preferred_element_type=jnp.float32)
    m = m + a_bc * h1_ref[...]
    h = _leaky(m / (den + 1e-16) + b1_ref[...])
    h2 = jnp.dot(h, w2_ref[...], preferred_element_type=jnp.float32)
    h2_ref[...] = h2
    als2_ref[...] = jnp.dot(h2, as2_ref[...], preferred_element_type=jnp.float32)
    ald2_ref[...] = jnp.dot(h2, ad2_ref[...], preferred_element_type=jnp.float32)


def _tc3_body(accm_ref, h2_ref, als2_ref, ald2_ref, c_ref, b2_ref, out_ref):
    t = accm_ref[0] + accm_ref[1]
    m = t[:, :EMB]
    d1 = t[:, EMB:]
    a_self = jnp.exp(_leaky(als2_ref[...] + ald2_ref[...]))
    den = jnp.dot(a_self, c_ref[...], preferred_element_type=jnp.float32) + d1
    a_bc = jnp.dot(a_self, c_ref[...], preferred_element_type=jnp.float32)
    m = m + a_bc * h2_ref[...]
    out_ref[...] = _leaky(m / (den + 1e-16) + b2_ref[...])


def _sc_body_factory(H, nb, kk):
    D = 16 * H
    PW = D + H
    K = kk
    nbuf = nb

    def body(*refs):
        ht, als, ald, srcp, dstp, zm = refs[:6]
        accm_o = refs[6]
        sc = list(refs[7:])

        def take(k):
            out = sc[:k]
            del sc[:k]
            return out

        (accm_sp,) = take(1)
        src = take(nbuf)
        dst = take(nbuf)
        dsts = take(nbuf)
        hB = take(nbuf)
        asB = take(nbuf)
        adB = take(nbuf)
        mB = take(nbuf)
        semg = take(nbuf)
        sems = take(nbuf)
        assert not sc

        cid = lax.axis_index("c")
        sid = lax.axis_index("s")
        wid = cid * NS + sid
        cpt = (srcp.shape[0] - nbuf * K) // (NW * K)

        stripe = pl.ds(sid * RPT, RPT)
        pltpu.sync_copy(zm, accm_sp.at[stripe])
        plsc.subcore_barrier()

        lane = lax.iota(jnp.int32, 16)
        head_mask = lane < H
        pidx = (lane + H) % 16
        base0 = wid * (cpt * K)

        def fetch(b, g):
            base = base0 + g * K
            pltpu.sync_copy(srcp.at[pl.ds(base, K)], src[b])
            pltpu.sync_copy(dstp.at[pl.ds(base, K)], dst[b])
            pltpu.async_copy(ht.at[src[b]], hB[b], semg[b])
            pltpu.async_copy(als.at[src[b]], asB[b], semg[b])
            pltpu.async_copy(ald.at[dst[b]], adB[b], semg[b])

        def wait_scat(b):
            pltpu.make_async_copy(mB[b], accm_sp.at[dsts[b]], sems[b]).wait()

        def half(g, b, first):
            hb, asb, adb, mb = hB[b], asB[b], adB[b], mB[b]
            pltpu.make_async_copy(ht.at[src[b]], hb, semg[b]).wait()
            pltpu.make_async_copy(als.at[src[b]], asb, semg[b]).wait()
            pltpu.make_async_copy(ald.at[dst[b]], adb, semg[b]).wait()
            if not first:
                wait_scat(b)

            def edge(e, c2):
                a = jnp.exp(_leaky(asb[e] + adb[e]))
                a = jnp.where(head_mask, a, 0.0)
                m_last = a
                for j in range(H):
                    bc = _permute(a, jnp.full((16,), j, jnp.int32))
                    m_last = bc * hb[e, pl.ds(16 * j, 16)]
                    mb[e, pl.ds(16 * j, 16)] = m_last
                tail = _permute(jnp.where(lane < H, a, m_last), pidx)
                mb[e, pl.ds(D + H - 16, 16)] = tail
                return c2

            lax.fori_loop(0, K, edge, 0)
            for i in range(K // 16):
                dsts[b][pl.ds(16 * i, 16)] = dst[b][pl.ds(16 * i, 16)]
            pltpu.async_copy(mb, accm_sp.at[dsts[b]], sems[b], add=True)
            fetch(b, g + nbuf)

        for b in range(nbuf):
            fetch(b, b)
        for b in range(nbuf):
            half(b, b, True)

        def grp(i2, carry):
            g = i2 * nbuf
            for b in range(nbuf):
                half(g + b, b, False)
            return carry

        lax.fori_loop(1, cpt // nbuf, grp, 0)

        for b in range(nbuf):
            wait_scat(b)
            pltpu.make_async_copy(ht.at[src[b]], hB[b], semg[b]).wait()
            pltpu.make_async_copy(als.at[src[b]], asB[b], semg[b]).wait()
            pltpu.make_async_copy(ald.at[dst[b]], adB[b], semg[b]).wait()

        plsc.subcore_barrier()
        pltpu.sync_copy(accm_sp.at[stripe], accm_o.at[cid, stripe])

    return body


_NBUF1, _K1 = 2, 64
_NBUF2, _K2 = 4, 128
_sc1_body = _sc_body_factory(H1, _NBUF1, _K1)
_sc2_body = _sc_body_factory(1, _NBUF2, _K2)


_SC_PARAMS = pltpu.CompilerParams(use_tc_tiling_on_sc=False)


def _sc_scratch(H, nbuf, K):
    D = 16 * H
    PW = D + H
    f32 = jnp.float32
    return (
        [pltpu.VMEM_SHARED((NP, PW), f32)]
        + [pltpu.VMEM((K,), jnp.int32) for _ in range(3 * nbuf)]
        + [pltpu.VMEM((K, D), f32) for _ in range(nbuf)]
        + [pltpu.VMEM((K, 16), f32) for _ in range(2 * nbuf)]
        + [pltpu.VMEM((K, PW), f32) for _ in range(nbuf)]
        + [pltpu.SemaphoreType.DMA for _ in range(2 * nbuf)]
    )


_sc1 = functools.partial(
    pl.kernel,
    out_type=jax.ShapeDtypeStruct((NC, NP, IN + H1), jnp.float32),
    mesh=_mesh,
    compiler_params=_SC_PARAMS,
    scratch_types=_sc_scratch(H1, _NBUF1, _K1),
)(_sc1_body)

_sc2 = functools.partial(
    pl.kernel,
    out_type=jax.ShapeDtypeStruct((NC, NP, EMB + 1), jnp.float32),
    mesh=_mesh,
    compiler_params=_SC_PARAMS,
    scratch_types=_sc_scratch(1, _NBUF2, _K2),
)(_sc2_body)


def kernel(x, edge_index, W1, a1_src, a1_dst, b1, W2, a2_src, a2_dst, b2):
    n = x.shape[0]
    e = edge_index.shape[1]

    def pad_edges(kk, nbuf):
        blk = NW * kk * nbuf
        pad = ((e + blk - 1) // blk) * blk - e + nbuf * kk
        srcp = jnp.concatenate(
            [edge_index[0], jnp.zeros((pad,), edge_index.dtype)])
        dstp = jnp.concatenate(
            [edge_index[1], jnp.full((pad,), n, edge_index.dtype)])
        return srcp, dstp

    srcp1, dstp1 = pad_edges(_K1, _NBUF1)
    srcp2, dstp2 = pad_edges(_K2, _NBUF2)

    f32 = jnp.float32
    cc = jnp.arange(IN)
    hh = jnp.arange(16)
    a1s_flat = a1_src.reshape(-1)
    a1d_flat = a1_dst.reshape(-1)
    blockdiag = (cc[:, None] // EMB == hh[None, :]).astype(f32)
    A1s = blockdiag * a1s_flat[:, None]
    A1d = blockdiag * a1d_flat[:, None]
    R = (jnp.arange(IN)[None, :] // EMB == jnp.arange(16)[:, None]).astype(f32)
    R8 = (jnp.arange(IN)[None, :] // EMB == jnp.arange(H1)[:, None]).astype(f32)
    A2s = jnp.zeros((EMB, 16), f32).at[:, 0].set(a2_src[0])
    A2d = jnp.zeros((EMB, 16), f32).at[:, 0].set(a2_dst[0])
    C = jnp.zeros((16, EMB), f32).at[0, :].set(1.0)

    xp = jnp.zeros((NP, IN), f32).at[:n].set(x)
    zm1 = jnp.zeros((RPT, IN + H1), f32)
    zm2 = jnp.zeros((RPT, EMB + 1), f32)

    BN = 2504
    G = NP // BN

    h1t, als, ald = pl.pallas_call(
        _tc1_body,
        grid=(G,),
        in_specs=[
            pl.BlockSpec((BN, IN), lambda i: (i, 0)),
            pl.BlockSpec((IN, IN), lambda i: (0, 0)),
            pl.BlockSpec((IN, 16), lambda i: (0, 0)),
            pl.BlockSpec((IN, 16), lambda i: (0, 0)),
        ],
        out_specs=[
            pl.BlockSpec((BN, IN), lambda i: (i, 0)),
            pl.BlockSpec((BN, 16), lambda i: (i, 0)),
            pl.BlockSpec((BN, 16), lambda i: (i, 0)),
        ],
        out_shape=[
            jax.ShapeDtypeStruct((NP, IN), f32),
            jax.ShapeDtypeStruct((NP, 16), f32),
            jax.ShapeDtypeStruct((NP, 16), f32),
        ],
    )(xp, W1, A1s, A1d)

    accm1 = _sc1(h1t, als, ald, srcp1, dstp1, zm1)

    h2t, als2, ald2 = pl.pallas_call(
        _tc2_body,
        grid=(G,),
        in_specs=[
            pl.BlockSpec((NC, BN, IN + H1), lambda i: (0, i, 0)),
            pl.BlockSpec((BN, IN), lambda i: (i, 0)),
            pl.BlockSpec((BN, 16), lambda i: (i, 0)),
            pl.BlockSpec((BN, 16), lambda i: (i, 0)),
            pl.BlockSpec((H1, IN), lambda i: (0, 0)),
            pl.BlockSpec((16, IN), lambda i: (0, 0)),
            pl.BlockSpec((1, IN), lambda i: (0, 0)),
            pl.BlockSpec((IN, EMB), lambda i: (0, 0)),
            pl.BlockSpec((EMB, 16), lambda i: (0, 0)),
            pl.BlockSpec((EMB, 16), lambda i: (0, 0)),
        ],
        out_specs=[
            pl.BlockSpec((BN, EMB), lambda i: (i, 0)),
            pl.BlockSpec((BN, 16), lambda i: (i, 0)),
            pl.BlockSpec((BN, 16), lambda i: (i, 0)),
        ],
        out_shape=[
            jax.ShapeDtypeStruct((NP, EMB), f32),
            jax.ShapeDtypeStruct((NP, 16), f32),
            jax.ShapeDtypeStruct((NP, 16), f32),
        ],
    )(accm1, h1t, als, ald, R8, R, b1.reshape(1, IN), W2, A2s, A2d)

    accm2 = _sc2(h2t, als2, ald2, srcp2, dstp2, zm2)

    BN3 = 1000
    out = pl.pallas_call(
        _tc3_body,
        grid=(n // BN3,),
        in_specs=[
            pl.BlockSpec((NC, BN3, EMB + 1), lambda i: (0, i, 0)),
            pl.BlockSpec((BN3, EMB), lambda i: (i, 0)),
            pl.BlockSpec((BN3, 16), lambda i: (i, 0)),
            pl.BlockSpec((BN3, 16), lambda i: (i, 0)),
            pl.BlockSpec((16, EMB), lambda i: (0, 0)),
            pl.BlockSpec((1, EMB), lambda i: (0, 0)),
        ],
        out_specs=pl.BlockSpec((BN3, EMB), lambda i: (i, 0)),
        out_shape=jax.ShapeDtypeStruct((n, EMB), f32),
    )(accm2, h2t, als2, ald2, C, b2.reshape(1, EMB))

    return out


# speedup vs baseline: 1.1622x; 1.1622x over previous
import functools

import jax
import jax.numpy as jnp
from jax import lax
from jax.experimental import pallas as pl
from jax.experimental.pallas import tpu as pltpu
from jax.experimental.pallas import tpu_sc as plsc

N = 10000
IN = 128
EMB = 16
H1 = 8
SLOPE = 0.2

NC = 2
NS = 16
NW = NC * NS
K = 64
RPT = 313
NP = NW * RPT

_mesh = plsc.VectorSubcoreMesh(
    core_axis_name="c", subcore_axis_name="s", num_cores=NC, num_subcores=NS)


def _leaky(v):
    return jnp.where(v > 0, v, v * SLOPE)


_GDN = lax.GatherDimensionNumbers(
    offset_dims=(), collapsed_slice_dims=(0,), start_index_map=(0,))


def _permute(vec, idx16):
    return lax.gather(vec, idx16[:, None], _GDN, (1,),
                      mode=lax.GatherScatterMode.PROMISE_IN_BOUNDS)


def _tc1_body(x_ref, w1_ref, as_ref, ad_ref, h1_ref, als_ref, ald_ref):
    h = jnp.dot(x_ref[...], w1_ref[...], preferred_element_type=jnp.float32)
    h1_ref[...] = h
    als_ref[...] = jnp.dot(h, as_ref[...], preferred_element_type=jnp.float32)
    ald_ref[...] = jnp.dot(h, ad_ref[...], preferred_element_type=jnp.float32)


def _tc2_body(accm_ref, accd_ref, h1_ref, als_ref, ald_ref, r_ref, b1_ref,
              w2_ref, as2_ref, ad2_ref, h2_ref, als2_ref, ald2_ref):
    m = accm_ref[0] + accm_ref[1]
    d = accd_ref[0] + accd_ref[1]
    a_self = jnp.exp(_leaky(als_ref[...] + ald_ref[...]))
    den = jnp.dot(d + a_self, r_ref[...], preferred_element_type=jnp.float32)
    a_bc = jnp.dot(a_self, r_ref[...], preferred_element_type=jnp.float32)
    m = m + a_bc * h1_ref[...]
    h = _leaky(m / (den + 1e-16) + b1_ref[...])
    h2 = jnp.dot(h, w2_ref[...], preferred_element_type=jnp.float32)
    h2_ref[...] = h2
    als2_ref[...] = jnp.dot(h2, as2_ref[...], preferred_element_type=jnp.float32)
    ald2_ref[...] = jnp.dot(h2, ad2_ref[...], preferred_element_type=jnp.float32)


def _tc3_body(accm_ref, accd_ref, h2_ref, als2_ref, ald2_ref, c_ref, b2_ref,
              out_ref):
    m = accm_ref[0] + accm_ref[1]
    d = accd_ref[0] + accd_ref[1]
    a_self = jnp.exp(_leaky(als2_ref[...] + ald2_ref[...]))
    den = jnp.dot(d + a_self, c_ref[...], preferred_element_type=jnp.float32)
    a_bc = jnp.dot(a_self, c_ref[...], preferred_element_type=jnp.float32)
    m = m + a_bc * h2_ref[...]
    out_ref[...] = _leaky(m / (den + 1e-16) + b2_ref[...])


def _sc_body_factory(H, nb, kk):
    D = 16 * H
    K = kk
    nbuf = nb

    def body(*refs):
        ht, als, ald, srcp, dstp, zm, zd = refs[:7]
        accm_o, accd_o = refs[7], refs[8]
        sc = list(refs[9:])

        def take(k):
            out = sc[:k]
            del sc[:k]
            return out

        (accm_sp,) = take(1)
        (accd_sp,) = take(1)
        src = take(nbuf)
        dst = take(nbuf)
        dsts = take(nbuf)
        hB = take(nbuf)
        asB = take(nbuf)
        adB = take(nbuf)
        alB = take(nbuf)
        mB = take(nbuf)
        semg = take(nbuf)
        sems = take(nbuf)
        assert not sc

        cid = lax.axis_index("c")
        sid = lax.axis_index("s")
        wid = cid * NS + sid
        cpt = (srcp.shape[0] - nbuf * K) // (NW * K)

        stripe = pl.ds(sid * RPT, RPT)
        pltpu.sync_copy(zm, accm_sp.at[stripe])
        pltpu.sync_copy(zd, accd_sp.at[stripe])
        plsc.subcore_barrier()

        lane = lax.iota(jnp.int32, 16)
        head_mask = lane < H
        base0 = wid * (cpt * K)

        def fetch(b, g):
            base = base0 + g * K
            pltpu.sync_copy(srcp.at[pl.ds(base, K)], src[b])
            pltpu.sync_copy(dstp.at[pl.ds(base, K)], dst[b])
            pltpu.async_copy(ht.at[src[b]], hB[b], semg[b])
            pltpu.async_copy(als.at[src[b]], asB[b], semg[b])
            pltpu.async_copy(ald.at[dst[b]], adB[b], semg[b])

        def wait_scat(b):
            pltpu.make_async_copy(mB[b], accm_sp.at[dsts[b]], sems[b]).wait()
            pltpu.make_async_copy(alB[b], accd_sp.at[dsts[b]], sems[b]).wait()

        def half(g, b, first):
            hb, asb, adb, alb, mb = hB[b], asB[b], adB[b], alB[b], mB[b]
            pltpu.make_async_copy(ht.at[src[b]], hb, semg[b]).wait()
            pltpu.make_async_copy(als.at[src[b]], asb, semg[b]).wait()
            pltpu.make_async_copy(ald.at[dst[b]], adb, semg[b]).wait()
            if not first:
                wait_scat(b)

            def edge(e, c2):
                a = jnp.exp(_leaky(asb[e] + adb[e]))
                a = jnp.where(head_mask, a, 0.0)
                alb[e] = a
                for j in range(H):
                    bc = _permute(a, jnp.full((16,), j, jnp.int32))
                    mb[e, pl.ds(16 * j, 16)] = bc * hb[e, pl.ds(16 * j, 16)]
                return c2

            lax.fori_loop(0, K, edge, 0)
            for i in range(K // 16):
                dsts[b][pl.ds(16 * i, 16)] = dst[b][pl.ds(16 * i, 16)]
            pltpu.async_copy(mb, accm_sp.at[dsts[b]], sems[b], add=True)
            pltpu.async_copy(alb, accd_sp.at[dsts[b]], sems[b], add=True)
            fetch(b, g + nbuf)

        for b in range(nbuf):
            fetch(b, b)
        for b in range(nbuf):
            half(b, b, True)

        def grp(i2, carry):
            g = i2 * nbuf
            for b in range(nbuf):
                half(g + b, b, False)
            return carry

        lax.fori_loop(1, cpt // nbuf, grp, 0)

        for b in range(nbuf):
            wait_scat(b)
            pltpu.make_async_copy(ht.at[src[b]], hB[b], semg[b]).wait()
            pltpu.make_async_copy(als.at[src[b]], asB[b], semg[b]).wait()
            pltpu.make_async_copy(ald.at[dst[b]], adB[b], semg[b]).wait()

        plsc.subcore_barrier()
        pltpu.sync_copy(accm_sp.at[stripe], accm_o.at[cid, stripe])
        pltpu.sync_copy(accd_sp.at[stripe], accd_o.at[cid, stripe])

    return body


_NBUF1, _K1 = 2, 64
_NBUF2, _K2 = 4, 128
_sc1_body = _sc_body_factory(H1, _NBUF1, _K1)
_sc2_body = _sc_body_factory(1, _NBUF2, _K2)


_SC_PARAMS = pltpu.CompilerParams(use_tc_tiling_on_sc=False)


def _sc_scratch(H, nbuf, K):
    D = 16 * H
    f32 = jnp.float32
    return (
        [pltpu.VMEM_SHARED((NP, D), f32)]
        + [pltpu.VMEM_SHARED((NP, 16), f32)]
        + [pltpu.VMEM((K,), jnp.int32) for _ in range(3 * nbuf)]
        + [pltpu.VMEM((K, D), f32) for _ in range(nbuf)]
        + [pltpu.VMEM((K, 16), f32) for _ in range(3 * nbuf)]
        + [pltpu.VMEM((K, D), f32) for _ in range(nbuf)]
        + [pltpu.SemaphoreType.DMA for _ in range(2 * nbuf)]
    )


_sc1 = functools.partial(
    pl.kernel,
    out_type=(jax.ShapeDtypeStruct((NC, NP, IN), jnp.float32),
              jax.ShapeDtypeStruct((NC, NP, 16), jnp.float32)),
    mesh=_mesh,
    compiler_params=_SC_PARAMS,
    scratch_types=_sc_scratch(H1, _NBUF1, _K1),
)(_sc1_body)

_sc2 = functools.partial(
    pl.kernel,
    out_type=(jax.ShapeDtypeStruct((NC, NP, EMB), jnp.float32),
              jax.ShapeDtypeStruct((NC, NP, 16), jnp.float32)),
    mesh=_mesh,
    compiler_params=_SC_PARAMS,
    scratch_types=_sc_scratch(1, _NBUF2, _K2),
)(_sc2_body)


def kernel(x, edge_index, W1, a1_src, a1_dst, b1, W2, a2_src, a2_dst, b2):
    n = x.shape[0]
    e = edge_index.shape[1]

    def pad_edges(kk, nbuf):
        blk = NW * kk * nbuf
        pad = ((e + blk - 1) // blk) * blk - e + nbuf * kk
        srcp = jnp.concatenate(
            [edge_index[0], jnp.zeros((pad,), edge_index.dtype)])
        dstp = jnp.concatenate(
            [edge_index[1], jnp.full((pad,), n, edge_index.dtype)])
        return srcp, dstp

    srcp1, dstp1 = pad_edges(_K1, _NBUF1)
    srcp2, dstp2 = pad_edges(_K2, _NBUF2)

    f32 = jnp.float32
    cc = jnp.arange(IN)
    hh = jnp.arange(16)
    a1s_flat = a1_src.reshape(-1)
    a1d_flat = a1_dst.reshape(-1)
    blockdiag = (cc[:, None] // EMB == hh[None, :]).astype(f32)
    A1s = blockdiag * a1s_flat[:, None]
    A1d = blockdiag * a1d_flat[:, None]
    R = (jnp.arange(IN)[None, :] // EMB == jnp.arange(16)[:, None]).astype(f32)
    R8 = (jnp.arange(IN)[None, :] // EMB == jnp.arange(H1)[:, None]).astype(f32)
    A2s = jnp.zeros((EMB, 16), f32).at[:, 0].set(a2_src[0])
    A2d = jnp.zeros((EMB, 16), f32).at[:, 0].set(a2_dst[0])
    C = jnp.zeros((16, EMB), f32).at[0, :].set(1.0)

    xp = jnp.zeros((NP, IN), f32).at[:n].set(x)
    zm = jnp.zeros((RPT, IN), f32)
    zm2 = jnp.zeros((RPT, EMB), f32)
    zd = jnp.zeros((RPT, 16), f32)

    BN = 2504
    G = NP // BN

    h1t, als, ald = pl.pallas_call(
        _tc1_body,
        grid=(G,),
        in_specs=[
            pl.BlockSpec((BN, IN), lambda i: (i, 0)),
            pl.BlockSpec((IN, IN), lambda i: (0, 0)),
            pl.BlockSpec((IN, 16), lambda i: (0, 0)),
            pl.BlockSpec((IN, 16), lambda i: (0, 0)),
        ],
        out_specs=[
            pl.BlockSpec((BN, IN), lambda i: (i, 0)),
            pl.BlockSpec((BN, 16), lambda i: (i, 0)),
            pl.BlockSpec((BN, 16), lambda i: (i, 0)),
        ],
        out_shape=[
            jax.ShapeDtypeStruct((NP, IN), f32),
            jax.ShapeDtypeStruct((NP, 16), f32),
            jax.ShapeDtypeStruct((NP, 16), f32),
        ],
    )(xp, W1, A1s, A1d)

    accm1, accd1 = _sc1(h1t, als, ald, srcp1, dstp1, zm, zd)

    h2t, als2, ald2 = pl.pallas_call(
        _tc2_body,
        grid=(G,),
        in_specs=[
            pl.BlockSpec((NC, BN, IN), lambda i: (0, i, 0)),
            pl.BlockSpec((NC, BN, 16), lambda i: (0, i, 0)),
            pl.BlockSpec((BN, IN), lambda i: (i, 0)),
            pl.BlockSpec((BN, 16), lambda i: (i, 0)),
            pl.BlockSpec((BN, 16), lambda i: (i, 0)),
            pl.BlockSpec((16, IN), lambda i: (0, 0)),
            pl.BlockSpec((1, IN), lambda i: (0, 0)),
            pl.BlockSpec((IN, EMB), lambda i: (0, 0)),
            pl.BlockSpec((EMB, 16), lambda i: (0, 0)),
            pl.BlockSpec((EMB, 16), lambda i: (0, 0)),
        ],
        out_specs=[
            pl.BlockSpec((BN, EMB), lambda i: (i, 0)),
            pl.BlockSpec((BN, 16), lambda i: (i, 0)),
            pl.BlockSpec((BN, 16), lambda i: (i, 0)),
        ],
        out_shape=[
            jax.ShapeDtypeStruct((NP, EMB), f32),
            jax.ShapeDtypeStruct((NP, 16), f32),
            jax.ShapeDtypeStruct((NP, 16), f32),
        ],
    )(accm1, accd1, h1t, als, ald, R, b1.reshape(1, IN), W2, A2s, A2d)

    accm2, accd2 = _sc2(h2t, als2, ald2, srcp2, dstp2, zm2, zd)

    BN3 = 1000
    out = pl.pallas_call(
        _tc3_body,
        grid=(n // BN3,),
        in_specs=[
            pl.BlockSpec((NC, BN3, EMB), lambda i: (0, i, 0)),
            pl.BlockSpec((NC, BN3, 16), lambda i: (0, i, 0)),
            pl.BlockSpec((BN3, EMB), lambda i: (i, 0)),
            pl.BlockSpec((BN3, 16), lambda i: (i, 0)),
            pl.BlockSpec((BN3, 16), lambda i: (i, 0)),
            pl.BlockSpec((16, EMB), lambda i: (0, 0)),
            pl.BlockSpec((1, EMB), lambda i: (0, 0)),
        ],
        out_specs=pl.BlockSpec((BN3, EMB), lambda i: (i, 0)),
        out_shape=jax.ShapeDtypeStruct((n, EMB), f32),
    )(accm2, accd2, h2t, als2, ald2, C, b2.reshape(1, EMB))

    return out
